# Initial kernel scaffold; baseline (speedup 1.0000x reference)
import functools

import jax
import jax.numpy as jnp
from jax import lax
from jax.experimental import pallas as pl
from jax.experimental.pallas import tpu as pltpu
from jax.experimental.pallas import tpu_sc as plsc

B = 4
N = 500000
D = 16
GX = GY = GZ = 64
GC = GX * GY * GZ
CHUNK = 65536
NCHUNK = GC // CHUNK
P = 2048
NBLK = (N + P - 1) // P
DUMP = 4096
GROWS = CHUNK + DUMP
ZROWS = 544


def _sc_body(locs_hbm, data_hbm, dens_hbm, out_hbm,
             locs_v, dens_v, data_v, cell_v, wgt_v, zero_v, grid_sh):
    c = lax.axis_index("c")
    s = lax.axis_index("s")
    lane = lax.iota(jnp.int32, 16)

    def _z(i, _):
        zero_v[i, :] = jnp.zeros((16,), jnp.float32)
        return 0
    lax.fori_loop(0, ZROWS, _z, 0, unroll=4)

    def _process_block(bid, b, my_chunk):
        s0 = b * N + bid * P
        cs = b * N + jnp.minimum(bid * P, N - P)
        pltpu.sync_copy(locs_hbm.at[pl.ds(cs, P), :], locs_v)
        pltpu.sync_copy(dens_hbm.at[pl.ds(cs, P)], dens_v)
        pltpu.sync_copy(data_hbm.at[pl.ds(cs, P), :], data_v)

        def _group(g, _):
            i16 = g * 16
            ivec = i16 + lane
            x = plsc.load_gather(locs_v, [ivec, jnp.zeros((16,), jnp.int32)])
            y = plsc.load_gather(locs_v, [ivec, jnp.full((16,), 1, jnp.int32)])
            z = plsc.load_gather(locs_v, [ivec, jnp.full((16,), 2, jnp.int32)])
            w = plsc.load_gather(locs_v, [ivec, jnp.full((16,), 3, jnp.int32)])
            dens = dens_v[pl.ds(i16, 16)]
            cx = jnp.clip((x * 64.0).astype(jnp.int32), 0, 63)
            cy = jnp.clip((y * 64.0).astype(jnp.int32), 0, 63)
            cz = jnp.clip((z * 64.0).astype(jnp.int32), 0, 63)
            flat = cx * 4096 + cy * 64 + cz
            valid = (cs + ivec) >= s0
            match = valid & (lax.shift_right_logical(flat, 16) == my_chunk)
            local = jnp.bitwise_and(flat, CHUNK - 1)
            dump = CHUNK + jnp.bitwise_and(ivec, DUMP - 1)
            cell = jnp.where(match, local, dump)
            r = g // 8
            col = (g % 8) * 16
            cell_v[r, pl.ds(col, 16)] = cell
            wgt_v[pl.ds(i16, 16)] = 1.0 / (w * dens)
            return 0
        lax.fori_loop(0, P // 16, _group, 0, unroll=2)

        def _scale(i, _):
            data_v[i, :] = data_v[i, :] * wgt_v[i]
            return 0
        lax.fori_loop(0, P, _scale, 0, unroll=8)

        pltpu.sync_copy(data_v, grid_sh.at[cell_v], add=True)
        return 0

    for b in range(B):
        for p in range(NCHUNK // 2):
            my_chunk = p * 2 + c
            for k in range(GROWS // (16 * ZROWS)):
                pltpu.sync_copy(
                    zero_v, grid_sh.at[pl.ds(s * (GROWS // 16) + k * ZROWS, ZROWS), :])
            plsc.subcore_barrier()

            def _blk(i, carry, b=b, my_chunk=my_chunk):
                bid = i * 16 + s

                @pl.when(bid < NBLK)
                def _():
                    _process_block(bid, b, my_chunk)
                return carry
            lax.fori_loop(0, 16, _blk, 0)
            plsc.subcore_barrier()

            base = b * GC + my_chunk * CHUNK + s * (CHUNK // 16)
            pltpu.sync_copy(grid_sh.at[pl.ds(s * (CHUNK // 16), CHUNK // 16), :],
                            out_hbm.at[pl.ds(base, CHUNK // 16), :])
            plsc.subcore_barrier()


@jax.jit
def _p2g(locs_f, data_f, dens_f):
    mesh = plsc.VectorSubcoreMesh(core_axis_name="c", subcore_axis_name="s")
    return pl.kernel(
        _sc_body,
        out_type=jax.ShapeDtypeStruct((B * GC, D), jnp.float32),
        mesh=mesh,
        scratch_types=[
            pltpu.VMEM((P, 4), jnp.float32),
            pltpu.VMEM((P,), jnp.float32),
            pltpu.VMEM((P, D), jnp.float32),
            pltpu.VMEM((16, 128), jnp.int32),
            pltpu.VMEM((P,), jnp.float32),
            pltpu.VMEM((ZROWS, D), jnp.float32),
            pltpu.VMEM_SHARED((GROWS, D), jnp.float32),
        ],
    )(locs_f, data_f, dens_f)


def kernel(locs, data, density):
    locs_f = locs.reshape(B * N, 4)
    data_f = data.reshape(B * N, D)
    dens_f = density.reshape(B * N)
    out = _p2g(locs_f, data_f, dens_f)
    return out.reshape(B, GX, GY, GZ, D)


# speedup vs baseline: 1.3204x; 1.3204x over previous
import functools

import jax
import jax.numpy as jnp
from jax import lax
from jax.experimental import pallas as pl
from jax.experimental.pallas import tpu as pltpu
from jax.experimental.pallas import tpu_sc as plsc

B = 4
N = 500000
D = 16
GX = GY = GZ = 64
GC = GX * GY * GZ
CHUNK = 65536
NCHUNK = GC // CHUNK
P = 2048
NBLK = (N + P - 1) // P
DUMP = 4096
GROWS = CHUNK + DUMP
ZROWS = 544


def _sc_body(locs_hbm, data_hbm, dens_hbm, out_hbm,
             locs_v, dens_v, data_v, cell_v, cell_row, wgt_v, zero_v, grid_sh):
    c = lax.axis_index("c")
    s = lax.axis_index("s")
    lane = lax.iota(jnp.int32, 16)

    def _z(i, _):
        zero_v[i, :] = jnp.zeros((16,), jnp.float32)
        return 0
    lax.fori_loop(0, ZROWS, _z, 0, unroll=4)

    def _process_block(bid, b, my_chunk):
        s0 = b * N + bid * P
        cs = b * N + jnp.minimum(bid * P, N - P)
        pltpu.sync_copy(locs_hbm.at[pl.ds(cs * 4, P * 4)], locs_v)
        pltpu.sync_copy(dens_hbm.at[pl.ds(cs, P)], dens_v)
        pltpu.sync_copy(data_hbm.at[pl.ds(cs, P), :], data_v)

        def _group(g, _):
            i16 = g * 16
            ivec = i16 + lane
            ivec4 = ivec * 4
            x = plsc.load_gather(locs_v, [ivec4])
            y = plsc.load_gather(locs_v, [ivec4 + 1])
            z = plsc.load_gather(locs_v, [ivec4 + 2])
            w = plsc.load_gather(locs_v, [ivec4 + 3])
            dens = dens_v[pl.ds(i16, 16)]
            cx = jnp.clip((x * 64.0).astype(jnp.int32), 0, 63)
            cy = jnp.clip((y * 64.0).astype(jnp.int32), 0, 63)
            cz = jnp.clip((z * 64.0).astype(jnp.int32), 0, 63)
            flat = cx * 4096 + cy * 64 + cz
            valid = (cs + ivec) >= s0
            match = valid & (lax.shift_right_logical(flat, 16) == my_chunk)
            local = jnp.bitwise_and(flat, CHUNK - 1)
            dump = CHUNK + jnp.bitwise_and(ivec, DUMP - 1)
            cell = jnp.where(match, local, dump)
            r = g // 8
            col = (g % 8) * 16
            cell_v[r, pl.ds(col, 16)] = cell
            wgt_v[pl.ds(i16, 16)] = 1.0 / (w * dens)
            return 0
        lax.fori_loop(0, P // 16, _group, 0, unroll=2)

        def _scale(g, _):
            wvec = wgt_v[pl.ds(g * 16, 16)]
            for j in range(16):
                i = g * 16 + j
                data_v[i, :] = data_v[i, :] * wvec[j]
            return 0
        lax.fori_loop(0, P // 16, _scale, 0)

        for k in range(16):
            for m in range(8):
                cell_row[pl.ds(m * 16, 16)] = cell_v[k, pl.ds(m * 16, 16)]
            pltpu.sync_copy(data_v.at[pl.ds(k * 128, 128), :],
                            grid_sh.at[cell_row], add=True)
        return 0

    for b in range(B):
        for p in range(NCHUNK // 2):
            my_chunk = p * 2 + c
            for k in range(GROWS // (16 * ZROWS)):
                pltpu.sync_copy(
                    zero_v, grid_sh.at[pl.ds(s * (GROWS // 16) + k * ZROWS, ZROWS), :])
            plsc.subcore_barrier()

            def _blk(i, carry, b=b, my_chunk=my_chunk):
                bid = i * 16 + s

                @pl.when(bid < NBLK)
                def _():
                    _process_block(bid, b, my_chunk)
                return carry
            lax.fori_loop(0, 16, _blk, 0)
            plsc.subcore_barrier()

            base = b * GC + my_chunk * CHUNK + s * (CHUNK // 16)
            pltpu.sync_copy(grid_sh.at[pl.ds(s * (CHUNK // 16), CHUNK // 16), :],
                            out_hbm.at[pl.ds(base, CHUNK // 16), :])
            plsc.subcore_barrier()


@jax.jit
def _p2g(locs_f, data_f, dens_f):
    mesh = plsc.VectorSubcoreMesh(core_axis_name="c", subcore_axis_name="s")
    return pl.kernel(
        _sc_body,
        out_type=jax.ShapeDtypeStruct((B * GC, D), jnp.float32),
        mesh=mesh,
        compiler_params=pltpu.CompilerParams(
            needs_layout_passes=False, use_tc_tiling_on_sc=False),
        scratch_types=[
            pltpu.VMEM((P * 4,), jnp.float32),
            pltpu.VMEM((P,), jnp.float32),
            pltpu.VMEM((P, D), jnp.float32),
            pltpu.VMEM((16, 128), jnp.int32),
            pltpu.VMEM((128,), jnp.int32),
            pltpu.VMEM((P,), jnp.float32),
            pltpu.VMEM((ZROWS, D), jnp.float32),
            pltpu.VMEM_SHARED((GROWS, D), jnp.float32),
        ],
    )(locs_f, data_f, dens_f)


def kernel(locs, data, density):
    locs_f = locs.reshape(B * N * 4)
    data_f = data.reshape(B * N, D)
    dens_f = density.reshape(B * N)
    out = _p2g(locs_f, data_f, dens_f)
    return out.reshape(B, GX, GY, GZ, D)


# speedup vs baseline: 1.3555x; 1.0266x over previous
import functools

import jax
import jax.numpy as jnp
from jax import lax
from jax.experimental import pallas as pl
from jax.experimental.pallas import tpu as pltpu
from jax.experimental.pallas import tpu_sc as plsc

B = 4
N = 500000
D = 16
GX = GY = GZ = 64
GC = GX * GY * GZ
CHUNK = 65536
NCHUNK = GC // CHUNK
P = 2048
NBLK = (N + P - 1) // P
DUMP = 4096
GROWS = CHUNK + DUMP
ZROWS = 544


def _sc_body(locs_hbm, data_hbm, dens_hbm, out_hbm,
             locs_v, dens_v, data_v, cell_v, wgt_v, zero_v, grid_sh):
    c = lax.axis_index("c")
    s = lax.axis_index("s")
    lane = lax.iota(jnp.int32, 16)

    def _z(i, _):
        zero_v[i, :] = jnp.zeros((16,), jnp.float32)
        return 0
    lax.fori_loop(0, ZROWS, _z, 0, unroll=4)

    def _process_block(bid, b, my_chunk):
        s0 = b * N + bid * P
        cs = b * N + jnp.minimum(bid * P, N - P)
        pltpu.sync_copy(locs_hbm.at[pl.ds(cs * 4, P * 4)], locs_v)
        pltpu.sync_copy(dens_hbm.at[pl.ds(cs, P)], dens_v)
        pltpu.sync_copy(data_hbm.at[pl.ds(cs, P), :], data_v)

        def _group(g, _):
            i16 = g * 16
            ivec = i16 + lane
            ivec4 = ivec * 4
            x = plsc.load_gather(locs_v, [ivec4])
            y = plsc.load_gather(locs_v, [ivec4 + 1])
            z = plsc.load_gather(locs_v, [ivec4 + 2])
            w = plsc.load_gather(locs_v, [ivec4 + 3])
            dens = dens_v[pl.ds(i16, 16)]
            cx = jnp.clip((x * 64.0).astype(jnp.int32), 0, 63)
            cy = jnp.clip((y * 64.0).astype(jnp.int32), 0, 63)
            cz = jnp.clip((z * 64.0).astype(jnp.int32), 0, 63)
            flat = cx * 4096 + cy * 64 + cz
            valid = (cs + ivec) >= s0
            match = valid & (lax.shift_right_logical(flat, 16) == my_chunk)
            local = jnp.bitwise_and(flat, CHUNK - 1)
            dump = CHUNK + jnp.bitwise_and(ivec, DUMP - 1)
            cell = jnp.where(match, local, dump)
            cell_v[pl.ds(i16, 16)] = cell
            wgt_v[pl.ds(i16, 16)] = 1.0 / (w * dens)
            return 0
        lax.fori_loop(0, P // 16, _group, 0, unroll=2)

        def _scale(g, _):
            wvec = wgt_v[pl.ds(g * 16, 16)]
            for j in range(16):
                i = g * 16 + j
                data_v[i, :] = data_v[i, :] * wvec[j]
            return 0
        lax.fori_loop(0, P // 16, _scale, 0)

        pltpu.sync_copy(data_v, grid_sh.at[cell_v], add=True)
        return 0

    for b in range(B):
        for p in range(NCHUNK // 2):
            my_chunk = p * 2 + c
            for k in range(GROWS // (16 * ZROWS)):
                pltpu.sync_copy(
                    zero_v, grid_sh.at[pl.ds(s * (GROWS // 16) + k * ZROWS, ZROWS), :])
            plsc.subcore_barrier()

            def _blk(i, carry, b=b, my_chunk=my_chunk):
                bid = i * 16 + s

                @pl.when(bid < NBLK)
                def _():
                    _process_block(bid, b, my_chunk)
                return carry
            lax.fori_loop(0, 16, _blk, 0)
            plsc.subcore_barrier()

            base = b * GC + my_chunk * CHUNK + s * (CHUNK // 16)
            pltpu.sync_copy(grid_sh.at[pl.ds(s * (CHUNK // 16), CHUNK // 16), :],
                            out_hbm.at[pl.ds(base, CHUNK // 16), :])
            plsc.subcore_barrier()


@jax.jit
def _p2g(locs_f, data_f, dens_f):
    mesh = plsc.VectorSubcoreMesh(core_axis_name="c", subcore_axis_name="s")
    return pl.kernel(
        _sc_body,
        out_type=jax.ShapeDtypeStruct((B * GC, D), jnp.float32),
        mesh=mesh,
        compiler_params=pltpu.CompilerParams(
            needs_layout_passes=False, use_tc_tiling_on_sc=False),
        scratch_types=[
            pltpu.VMEM((P * 4,), jnp.float32),
            pltpu.VMEM((P,), jnp.float32),
            pltpu.VMEM((P, D), jnp.float32),
            pltpu.VMEM((P,), jnp.int32),
            pltpu.VMEM((P,), jnp.float32),
            pltpu.VMEM((ZROWS, D), jnp.float32),
            pltpu.VMEM_SHARED((GROWS, D), jnp.float32),
        ],
    )(locs_f, data_f, dens_f)


def kernel(locs, data, density):
    locs_f = locs.reshape(B * N * 4)
    data_f = data.reshape(B * N, D)
    dens_f = density.reshape(B * N)
    out = _p2g(locs_f, data_f, dens_f)
    return out.reshape(B, GX, GY, GZ, D)


# speedup vs baseline: 1.4860x; 1.0962x over previous
import functools

import jax
import jax.numpy as jnp
from jax import lax
from jax.experimental import pallas as pl
from jax.experimental.pallas import tpu as pltpu
from jax.experimental.pallas import tpu_sc as plsc

B = 4
N = 500000
D = 16
GX = GY = GZ = 64
GC = GX * GY * GZ
CHUNK = 65536
NCHUNK = GC // CHUNK
NROUND = B * (NCHUNK // 2)
P = 1024
NBLK = (N + P - 1) // P
STEPS = (NBLK + 15) // 16
DUMP = 4096
GROWS = CHUNK + DUMP
TSLICE = GROWS // 16


def _sc_body(locs_hbm, data_hbm, dens_hbm, out_hbm,
             locs_v0, locs_v1, dens_v0, dens_v1, data_v0, data_v1,
             cell_v0, cell_v1, wgt_v, grid_sh,
             insem0, insem1, scsem0, scsem1):
    c = lax.axis_index("c")
    s = lax.axis_index("s")
    lane = lax.iota(jnp.int32, 16)
    locs_v = (locs_v0, locs_v1)
    dens_v = (dens_v0, dens_v1)
    data_v = (data_v0, data_v1)
    cell_v = (cell_v0, cell_v1)
    insem = (insem0, insem1)
    scsem = (scsem0, scsem1)

    def block_start(it, b):
        return b * N + jnp.minimum((it * 16 + s) * P, N - P)

    def compute(it, sl, b, my_chunk):
        s0 = b * N + (it * 16 + s) * P
        cs = block_start(it, b)
        lv, dv, dav, cv = locs_v[sl], dens_v[sl], data_v[sl], cell_v[sl]

        def _group(g, _):
            i16 = g * 16
            ivec = i16 + lane
            ivec4 = ivec * 4
            x = plsc.load_gather(lv, [ivec4])
            y = plsc.load_gather(lv, [ivec4 + 1])
            z = plsc.load_gather(lv, [ivec4 + 2])
            w = plsc.load_gather(lv, [ivec4 + 3])
            dens = dv[pl.ds(i16, 16)]
            cx = jnp.clip((x * 64.0).astype(jnp.int32), 0, 63)
            cy = jnp.clip((y * 64.0).astype(jnp.int32), 0, 63)
            cz = jnp.clip((z * 64.0).astype(jnp.int32), 0, 63)
            flat = cx * 4096 + cy * 64 + cz
            valid = (cs + ivec) >= s0
            match = valid & (lax.shift_right_logical(flat, 16) == my_chunk)
            local = jnp.bitwise_and(flat, CHUNK - 1)
            dump = CHUNK + jnp.bitwise_and(ivec, DUMP - 1)
            cv[pl.ds(i16, 16)] = jnp.where(match, local, dump)
            wgt_v[pl.ds(i16, 16)] = 1.0 / (w * dens)
            return 0
        lax.fori_loop(0, P // 16, _group, 0, unroll=2)

        def _scale(g, _):
            wvec = wgt_v[pl.ds(g * 16, 16)]
            for j in range(16):
                i = g * 16 + j
                dav[i, :] = dav[i, :] * wvec[j]
            return 0
        lax.fori_loop(0, P // 16, _scale, 0)

    def round_body(r, _):
        b = r // 2
        my_chunk = (r % 2) * 2 + c

        def _z(i, _):
            data_v0[i, :] = jnp.zeros((16,), jnp.float32)
            return 0
        lax.fori_loop(0, P, _z, 0, unroll=4)
        zb = s * TSLICE
        for k in range(TSLICE // P):
            pltpu.sync_copy(data_v0, grid_sh.at[pl.ds(zb + k * P, P), :])
        rem = TSLICE - (TSLICE // P) * P
        if rem:
            pltpu.sync_copy(data_v0.at[pl.ds(0, rem), :],
                            grid_sh.at[pl.ds(zb + (TSLICE // P) * P, rem), :])
        plsc.subcore_barrier()

        def active(it):
            return (it * 16 + s) < NBLK

        def start_in(it, sl):
            @pl.when(active(it))
            def _():
                cs = block_start(it, b)
                pltpu.async_copy(locs_hbm.at[pl.ds(cs * 4, P * 4)],
                                 locs_v[sl], insem[sl])
                pltpu.async_copy(dens_hbm.at[pl.ds(cs, P)],
                                 dens_v[sl], insem[sl])
                pltpu.async_copy(data_hbm.at[pl.ds(cs, P), :],
                                 data_v[sl], insem[sl])

        def wait_in(it, sl):
            @pl.when(active(it))
            def _():
                pltpu.make_async_copy(locs_hbm.at[pl.ds(0, P * 4)],
                                      locs_v[sl], insem[sl]).wait()
                pltpu.make_async_copy(dens_hbm.at[pl.ds(0, P)],
                                      dens_v[sl], insem[sl]).wait()
                pltpu.make_async_copy(data_hbm.at[pl.ds(0, P), :],
                                      data_v[sl], insem[sl]).wait()

        sc_descs = {}

        def fire_sc(it, sl):
            @pl.when(active(it))
            def _():
                sc_descs[it] = pltpu.async_copy(
                    data_v[sl], grid_sh.at[cell_v[sl]], scsem[sl], add=True)

        def wait_sc(it):
            @pl.when(active(it))
            def _():
                sc_descs[it].wait()

        start_in(0, 0)
        for it in range(STEPS):
            sl = it % 2
            if it + 1 < STEPS:
                if it >= 1:
                    wait_sc(it - 1)
                start_in(it + 1, sl ^ 1)
            wait_in(it, sl)

            @pl.when(active(it))
            def _(it=it, sl=sl):
                compute(it, sl, b, my_chunk)
            fire_sc(it, sl)
        wait_sc(STEPS - 2)
        wait_sc(STEPS - 1)
        plsc.subcore_barrier()

        base = b * GC + my_chunk * CHUNK + s * (CHUNK // 16)
        pltpu.sync_copy(grid_sh.at[pl.ds(s * (CHUNK // 16), CHUNK // 16), :],
                        out_hbm.at[pl.ds(base, CHUNK // 16), :])
        plsc.subcore_barrier()
        return 0

    lax.fori_loop(0, NROUND, round_body, 0)


@jax.jit
def _p2g(locs_f, data_f, dens_f):
    mesh = plsc.VectorSubcoreMesh(core_axis_name="c", subcore_axis_name="s")
    return pl.kernel(
        _sc_body,
        out_type=jax.ShapeDtypeStruct((B * GC, D), jnp.float32),
        mesh=mesh,
        compiler_params=pltpu.CompilerParams(
            needs_layout_passes=False, use_tc_tiling_on_sc=False),
        scratch_types=[
            pltpu.VMEM((P * 4,), jnp.float32),
            pltpu.VMEM((P * 4,), jnp.float32),
            pltpu.VMEM((P,), jnp.float32),
            pltpu.VMEM((P,), jnp.float32),
            pltpu.VMEM((P, D), jnp.float32),
            pltpu.VMEM((P, D), jnp.float32),
            pltpu.VMEM((P,), jnp.int32),
            pltpu.VMEM((P,), jnp.int32),
            pltpu.VMEM((P,), jnp.float32),
            pltpu.VMEM_SHARED((GROWS, D), jnp.float32),
            pltpu.SemaphoreType.DMA,
            pltpu.SemaphoreType.DMA,
            pltpu.SemaphoreType.DMA,
            pltpu.SemaphoreType.DMA,
        ],
    )(locs_f, data_f, dens_f)


def kernel(locs, data, density):
    locs_f = locs.reshape(B * N * 4)
    data_f = data.reshape(B * N, D)
    dens_f = density.reshape(B * N)
    out = _p2g(locs_f, data_f, dens_f)
    return out.reshape(B, GX, GY, GZ, D)
